# baseline (speedup 1.0000x reference)
import functools

import numpy as np
import jax
import jax.numpy as jnp
from jax import lax
from jax.experimental import pallas as pl
from jax.experimental.pallas import tpu as pltpu
from jax.experimental.pallas import tpu_sc as plsc

NC = 2
NS = 16
LANES = 16
NW = NC * NS
CH = 96


def _hi_lo(x):
    xh = x.astype(jnp.bfloat16)
    xl = (x - xh.astype(jnp.float32)).astype(jnp.bfloat16)
    return xh, xl


def _stack3(w):
    wh, wl = _hi_lo(w)
    return jnp.concatenate([wh, wl, wh], axis=0)


def _bf16_round_bits(x):
    b = jax.lax.bitcast_convert_type(x, jnp.int32)
    return jax.lax.shift_right_logical(b + jnp.int32(32768), 16)


def _pack_rows_bf16(w, D):
    bits = _bf16_round_bits(w)
    return jax.lax.bitwise_or(
        bits[:, :D // 2], jax.lax.shift_left(bits[:, D // 2:], 16))


def _edge_weights_tc(es, ea, w1stk, w2stk, L, D):
    E, S = es.shape
    B = 512
    inv = 1.0 / np.sqrt(S)

    def body(es_ref, ea_ref, w1_ref, w2_ref, *out_refs):
        esb = es_ref[...]
        eab = ea_ref[...]
        esh, esl = _hi_lo(esb)
        lhs1 = jnp.concatenate([esh, esh, esl], axis=1)
        hcat = jax.nn.silu(
            jnp.dot(lhs1, w1_ref[...], preferred_element_type=jnp.float32)
            * inv)
        for l in range(L):
            hh, hl = _hi_lo(hcat[:, S * l:S * (l + 1)])
            lhs2 = jnp.concatenate([hh, hh, hl], axis=1)
            w = jnp.dot(lhs2, w2_ref[l],
                        preferred_element_type=jnp.float32) * inv
            out_refs[l][...] = _pack_rows_bf16(w * eab, D)

    return pl.pallas_call(
        body,
        grid=(E // B,),
        in_specs=[
            pl.BlockSpec((B, S), lambda e: (e, 0)),
            pl.BlockSpec((B, 1), lambda e: (e, 0)),
            pl.BlockSpec((3 * S, L * S), lambda e: (0, 0)),
            pl.BlockSpec((L, 3 * S, D), lambda e: (0, 0, 0)),
        ],
        out_specs=[pl.BlockSpec((B, D // 2), lambda e: (e, 0))] * L,
        out_shape=[jax.ShapeDtypeStruct((E, D // 2), jnp.int32)] * L,
    )(es, ea, w1stk, w2stk)


def _node_linears_tc(x, na, lin1, scw):
    N, D = x.shape
    B = 1000
    inv = 1.0 / np.sqrt(D)

    def body(x_ref, na_ref, l1_ref, sc_ref, nf_ref, scv_ref):
        xa = x_ref[...]
        naa = na_ref[...]
        xh, xl = _hi_lo(xa)
        lhs = jnp.concatenate([xh, xh, xl], axis=1)
        nf_ref[...] = jnp.dot(lhs, l1_ref[...],
                              preferred_element_type=jnp.float32) * inv * naa
        scv_ref[...] = jnp.dot(lhs, sc_ref[...],
                               preferred_element_type=jnp.float32) * inv * naa

    return pl.pallas_call(
        body,
        grid=(N // B,),
        in_specs=[
            pl.BlockSpec((B, D), lambda i: (i, 0)),
            pl.BlockSpec((B, 1), lambda i: (i, 0)),
            pl.BlockSpec((3 * D, D), lambda i: (0, 0)),
            pl.BlockSpec((3 * D, D), lambda i: (0, 0)),
        ],
        out_specs=[pl.BlockSpec((B, D), lambda i: (i, 0))] * 2,
        out_shape=[jax.ShapeDtypeStruct((N, D), jnp.float32)] * 2,
    )(x, na, lin1, scw)


def _combine_tc(aggp, cntp, scv, na, lin2, l3, apply_silu):
    N, D = scv.shape
    B = 1000
    inv = 1.0 / np.sqrt(D)

    def body(agg_ref, cnt_ref, scv_ref, na_ref, l2_ref, l3_ref, out_ref):
        cnt = cnt_ref[0, :, 0:1] + cnt_ref[1, :, 0:1]
        a = (agg_ref[0] + agg_ref[1]) / jnp.sqrt(cnt)
        naa = na_ref[...]
        co = jnp.dot(a, l2_ref[...], preferred_element_type=jnp.float32) * inv * naa
        ang = 0.1 * jnp.dot(a, l3_ref[...],
                            preferred_element_type=jnp.float32) * inv * naa
        out = jnp.cos(ang) * scv_ref[...] + jnp.sin(ang) * co
        if apply_silu:
            out = out * jax.nn.sigmoid(out)
        out_ref[...] = out

    return pl.pallas_call(
        body,
        grid=(N // B,),
        in_specs=[
            pl.BlockSpec((2, B, D), lambda i: (0, i, 0)),
            pl.BlockSpec((2, B, LANES), lambda i: (0, i, 0)),
            pl.BlockSpec((B, D), lambda i: (i, 0)),
            pl.BlockSpec((B, 1), lambda i: (i, 0)),
            pl.BlockSpec((D, D), lambda i: (0, 0)),
            pl.BlockSpec((D, 1), lambda i: (0, 0)),
        ],
        out_specs=pl.BlockSpec((B, D), lambda i: (i, 0)),
        out_shape=jax.ShapeDtypeStruct((N, D), jnp.float32),
    )(aggp, cntp, scv, na, lin2, l3)


def _make_sc_kernel(N_pad, E_pad, D):
    TOT = E_pad // CH
    nchk = TOT // NW
    nquad = nchk // 4
    rps = N_pad // NS
    mesh = plsc.VectorSubcoreMesh(core_axis_name="c", subcore_axis_name="s")

    out_type = [jax.ShapeDtypeStruct((NC, N_pad, D), jnp.float32)]
    scratch = [
        [pltpu.VMEM((2, CH), jnp.int32) for _ in range(4)],
        [pltpu.VMEM((CH, D), jnp.float32) for _ in range(2)],
        [pltpu.VMEM((CH, D // 2), jnp.int32) for _ in range(2)],
        pltpu.VMEM_SHARED((N_pad, D), jnp.float32),
        [pltpu.SemaphoreType.DMA for _ in range(4)],
        [pltpu.SemaphoreType.DMA for _ in range(2)],
        [pltpu.SemaphoreType.DMA for _ in range(2)],
        [pltpu.SemaphoreType.DMA for _ in range(2)],
    ]

    def body(nf_hbm, we_hbm, idx_hbm, zrows_hbm, agg_out,
             ibuf, rows, web, agg_sp, isem, gsem, wsem, ssem):
        c = lax.axis_index("c")
        s = lax.axis_index("s")
        wid = s * NC + c
        base = wid * nchk

        pltpu.sync_copy(zrows_hbm.at[pl.ds(s * rps, rps)],
                        agg_sp.at[pl.ds(s * rps, rps)])
        plsc.subcore_barrier()

        def start_idx(j, r):
            pltpu.async_copy(idx_hbm.at[base + j], ibuf[r], isem[r])

        def wait_idx(j, r):
            pltpu.make_async_copy(idx_hbm.at[base + j], ibuf[r], isem[r]).wait()

        def start_gw(j, r, q):
            pltpu.async_copy(nf_hbm.at[ibuf[r].at[0]], rows[q], gsem[q])
            pltpu.async_copy(we_hbm.at[base + j], web[q], wsem[q])

        def wait_gw(j, r, q):
            pltpu.make_async_copy(nf_hbm.at[ibuf[r].at[0]], rows[q],
                                  gsem[q]).wait()
            pltpu.make_async_copy(we_hbm.at[base + j], web[q], wsem[q]).wait()

        himask = jnp.int32(-65536)

        def mult(q):
            def mul(j, _):
                rr = j // (D // 32)
                c16 = (j % (D // 32)) * LANES
                bc = lambda v: lax.bitcast_convert_type(v, jnp.float32)
                wi = web[q][rr, pl.ds(c16, LANES)]
                wa = bc(lax.shift_left(wi, 16))
                wb = bc(lax.bitwise_and(wi, himask))
                rows[q][rr, pl.ds(c16, LANES)] = (
                    rows[q][rr, pl.ds(c16, LANES)] * wa)
                rows[q][rr, pl.ds(c16 + D // 2, LANES)] = (
                    rows[q][rr, pl.ds(c16 + D // 2, LANES)] * wb)
                return 0
            lax.fori_loop(0, CH * (D // 32), mul, 0, unroll=8)

        def start_scat(r, q):
            pltpu.async_copy(rows[q], agg_sp.at[ibuf[r].at[1]], ssem[q],
                             add=True)

        def wait_scat(r, q):
            pltpu.make_async_copy(rows[q], agg_sp.at[ibuf[r].at[1]],
                                  ssem[q]).wait()

        start_idx(0, 0)
        start_idx(1, 1)
        wait_idx(0, 0)
        start_gw(0, 0, 0)
        start_idx(2, 2)
        wait_gw(0, 0, 0)
        wait_idx(1, 1)
        start_gw(1, 1, 1)
        mult(0)
        start_scat(0, 0)
        start_idx(3, 3)
        wait_gw(1, 1, 1)
        wait_scat(0, 0)
        wait_idx(2, 2)
        start_gw(2, 2, 0)
        mult(1)
        start_scat(1, 1)

        def quad_body(i, carry):
            for rr in range(4):
                j = 4 * i + 2 + rr
                slot = (2 + rr) % 4
                q = rr % 2
                start_idx(j + 2, (slot + 2) % 4)
                wait_gw(j, slot, q)
                wait_scat((slot + 3) % 4, 1 - q)
                wait_idx(j + 1, (slot + 1) % 4)
                start_gw(j + 1, (slot + 1) % 4, 1 - q)
                mult(q)
                start_scat(slot, q)
            return carry

        lax.fori_loop(0, (nchk - 4) // 4, quad_body, 0)
        wait_gw(nchk - 2, 2, 0)
        wait_scat(1, 1)
        wait_idx(nchk - 1, 3)
        start_gw(nchk - 1, 3, 1)
        mult(0)
        start_scat(2, 0)
        wait_gw(nchk - 1, 3, 1)
        wait_scat(2, 0)
        mult(1)
        start_scat(3, 1)
        wait_scat(3, 1)
        plsc.subcore_barrier()

        pltpu.sync_copy(agg_sp.at[pl.ds(s * rps, rps)],
                        agg_out.at[c, pl.ds(s * rps, rps)])

    return pl.kernel(body, out_type=out_type, mesh=mesh, scratch_types=scratch)


def _make_count_kernel(N_pad, E_pad):
    n_chunks = E_pad // CH // NW
    rps = N_pad // NS
    mesh = plsc.VectorSubcoreMesh(core_axis_name="c", subcore_axis_name="s")

    out_type = [jax.ShapeDtypeStruct((NC, N_pad, LANES), jnp.float32)]
    scratch = [
        pltpu.VMEM((CH,), jnp.int32),
        pltpu.VMEM((CH, LANES), jnp.float32),
        pltpu.VMEM_SHARED((N_pad, LANES), jnp.float32),
    ]

    def body(dst_hbm, zcnt_hbm, ones_hbm, cnt_out, dst_v, ones_v, cnt_sp):
        c = lax.axis_index("c")
        s = lax.axis_index("s")
        wid = s * NC + c
        pltpu.sync_copy(zcnt_hbm.at[pl.ds(s * rps, rps)],
                        cnt_sp.at[pl.ds(s * rps, rps)])
        pltpu.sync_copy(ones_hbm, ones_v)
        plsc.subcore_barrier()
        base = wid * n_chunks

        def chunk(i, carry):
            pltpu.sync_copy(dst_hbm.at[base + i], dst_v)
            pltpu.sync_copy(ones_v, cnt_sp.at[dst_v], add=True)
            return carry

        lax.fori_loop(0, n_chunks, chunk, 0)
        plsc.subcore_barrier()
        pltpu.sync_copy(cnt_sp.at[pl.ds(s * rps, rps)],
                        cnt_out.at[c, pl.ds(s * rps, rps)])

    return pl.kernel(body, out_type=out_type, mesh=mesh, scratch_types=scratch)


def kernel(node_features, node_attr, edge_src, edge_dst, edge_attr,
           edge_scalars, params):
    N, D = node_features.shape
    E = edge_scalars.shape[0]
    S = edge_scalars.shape[1]
    quantum = 4 * NW * CH
    E_pad = ((E + quantum - 1) // quantum) * quantum
    N_pad = ((N + 1 + 127) // 128) * 128
    pad = E_pad - E
    if pad:
        src_p = jnp.concatenate([edge_src, jnp.zeros((pad,), jnp.int32)])
        dst_p = jnp.concatenate([edge_dst, jnp.full((pad,), N, jnp.int32)])
        es_p = jnp.concatenate([edge_scalars, jnp.zeros((pad, S), jnp.float32)])
        ea_p = jnp.concatenate([edge_attr, jnp.zeros((pad, 1), jnp.float32)])
    else:
        src_p, dst_p, es_p, ea_p = edge_src, edge_dst, edge_scalars, edge_attr

    L = len(params)

    w1stk = _stack3(jnp.concatenate([p['fc_w1'] for p in params], axis=1))
    w2stk = jnp.stack([_stack3(p['fc_w2']) for p in params])
    we_all = _edge_weights_tc(es_p, ea_p, w1stk, w2stk, L, D)

    TOT = E_pad // CH
    src2 = src_p.reshape(TOT, CH)
    dst2 = dst_p.reshape(TOT, CH)
    idx2 = jnp.stack([src2, dst2], axis=1)
    we3 = [w.reshape(TOT, CH, D // 2) for w in we_all]

    zrows = jnp.zeros((N_pad, D), jnp.float32)
    zcnt = jnp.zeros((N_pad, LANES), jnp.float32)
    ones_b = jnp.ones((CH, LANES), jnp.float32)

    sc_edge = _make_sc_kernel(N_pad, E_pad, D)
    (cntp,) = _make_count_kernel(N_pad, E_pad)(dst2, zcnt, ones_b)

    x = node_features
    for li, p in enumerate(params):
        nf, scv = _node_linears_tc(x, node_attr,
                                   _stack3(p['lin1_w'][:, 0, :]),
                                   _stack3(p['sc_w'][:, 0, :]))
        (aggp,) = sc_edge(nf, we3[li], idx2, zrows)
        x = _combine_tc(aggp, cntp, scv, node_attr, p['lin2_w'][:, 0, :],
                        p['lin3_w'][:, 0, :], li < L - 1)
    return x


# speedup vs baseline: 1.4409x; 1.4409x over previous
import functools

import numpy as np
import jax
import jax.numpy as jnp
from jax import lax
from jax.experimental import pallas as pl
from jax.experimental.pallas import tpu as pltpu
from jax.experimental.pallas import tpu_sc as plsc

NC = 2
NS = 16
LANES = 16
NW = NC * NS
CH = 56


def _hi_lo(x):
    xh = x.astype(jnp.bfloat16)
    xl = (x - xh.astype(jnp.float32)).astype(jnp.bfloat16)
    return xh, xl


def _stack3(w):
    wh, wl = _hi_lo(w)
    return jnp.concatenate([wh, wl, wh], axis=0)


def _bf16_round_bits(x):
    b = jax.lax.bitcast_convert_type(x, jnp.int32)
    return jax.lax.shift_right_logical(b + jnp.int32(32768), 16)


def _pack_rows_bf16(w, D):
    bits = _bf16_round_bits(w)
    return jax.lax.bitwise_or(
        bits[:, :D // 2], jax.lax.shift_left(bits[:, D // 2:], 16))


def _edge_weights_tc(es, ea, w1stk, w2stk, L, D):
    E, S = es.shape
    B = 512
    inv = 1.0 / np.sqrt(S)

    def body(es_ref, ea_ref, w1_ref, w2_ref, *out_refs):
        esb = es_ref[...]
        eab = ea_ref[...]
        esh, esl = _hi_lo(esb)
        lhs1 = jnp.concatenate([esh, esh, esl], axis=1)
        hcat = jax.nn.silu(
            jnp.dot(lhs1, w1_ref[...], preferred_element_type=jnp.float32)
            * inv)
        for l in range(L):
            hh, hl = _hi_lo(hcat[:, S * l:S * (l + 1)])
            lhs2 = jnp.concatenate([hh, hh, hl], axis=1)
            w = jnp.dot(lhs2, w2_ref[l],
                        preferred_element_type=jnp.float32) * inv
            out_refs[l][...] = _pack_rows_bf16(w * eab, D)

    return pl.pallas_call(
        body,
        grid=(E // B,),
        in_specs=[
            pl.BlockSpec((B, S), lambda e: (e, 0)),
            pl.BlockSpec((B, 1), lambda e: (e, 0)),
            pl.BlockSpec((3 * S, L * S), lambda e: (0, 0)),
            pl.BlockSpec((L, 3 * S, D), lambda e: (0, 0, 0)),
        ],
        out_specs=[pl.BlockSpec((B, D // 2), lambda e: (e, 0))] * L,
        out_shape=[jax.ShapeDtypeStruct((E, D // 2), jnp.int32)] * L,
    )(es, ea, w1stk, w2stk)


def _node_linears_tc(x, na, lin1, scw):
    N, D = x.shape
    B = 1000
    inv = 1.0 / np.sqrt(D)

    def body(x_ref, na_ref, l1_ref, sc_ref, nf_ref, scv_ref):
        xa = x_ref[...]
        naa = na_ref[...]
        xh, xl = _hi_lo(xa)
        lhs = jnp.concatenate([xh, xh, xl], axis=1)
        nf_ref[...] = jnp.dot(lhs, l1_ref[...],
                              preferred_element_type=jnp.float32) * inv * naa
        scv_ref[...] = jnp.dot(lhs, sc_ref[...],
                               preferred_element_type=jnp.float32) * inv * naa

    return pl.pallas_call(
        body,
        grid=(N // B,),
        in_specs=[
            pl.BlockSpec((B, D), lambda i: (i, 0)),
            pl.BlockSpec((B, 1), lambda i: (i, 0)),
            pl.BlockSpec((3 * D, D), lambda i: (0, 0)),
            pl.BlockSpec((3 * D, D), lambda i: (0, 0)),
        ],
        out_specs=[pl.BlockSpec((B, D), lambda i: (i, 0))] * 2,
        out_shape=[jax.ShapeDtypeStruct((N, D), jnp.float32)] * 2,
    )(x, na, lin1, scw)


def _combine_tc(aggp, cntp, scv, na, lin2, l3, apply_silu):
    N, D = scv.shape
    B = 1000
    inv = 1.0 / np.sqrt(D)

    def body(agg_ref, cnt_ref, scv_ref, na_ref, l2_ref, l3_ref, out_ref):
        cnt = cnt_ref[0, :, 0:1] + cnt_ref[1, :, 0:1]
        a = (agg_ref[0] + agg_ref[1]) / jnp.sqrt(cnt)
        naa = na_ref[...]
        co = jnp.dot(a, l2_ref[...], preferred_element_type=jnp.float32) * inv * naa
        ang = 0.1 * jnp.dot(a, l3_ref[...],
                            preferred_element_type=jnp.float32) * inv * naa
        out = jnp.cos(ang) * scv_ref[...] + jnp.sin(ang) * co
        if apply_silu:
            out = out * jax.nn.sigmoid(out)
        out_ref[...] = out

    return pl.pallas_call(
        body,
        grid=(N // B,),
        in_specs=[
            pl.BlockSpec((2, B, D), lambda i: (0, i, 0)),
            pl.BlockSpec((2, B, LANES), lambda i: (0, i, 0)),
            pl.BlockSpec((B, D), lambda i: (i, 0)),
            pl.BlockSpec((B, 1), lambda i: (i, 0)),
            pl.BlockSpec((D, D), lambda i: (0, 0)),
            pl.BlockSpec((D, 1), lambda i: (0, 0)),
        ],
        out_specs=pl.BlockSpec((B, D), lambda i: (i, 0)),
        out_shape=jax.ShapeDtypeStruct((N, D), jnp.float32),
    )(aggp, cntp, scv, na, lin2, l3)


def _make_sc_kernel(N_pad, E_pad, D):
    TOT = E_pad // CH
    nchk = TOT // NW
    nquad = nchk // 4
    rps = N_pad // NS
    mesh = plsc.VectorSubcoreMesh(core_axis_name="c", subcore_axis_name="s")

    out_type = [jax.ShapeDtypeStruct((NC, N_pad, D), jnp.float32)]
    scratch = [
        [pltpu.VMEM((CH,), jnp.int32) for _ in range(4)],
        [pltpu.VMEM((CH,), jnp.int32) for _ in range(4)],
        [pltpu.VMEM((CH, D), jnp.float32) for _ in range(2)],
        [pltpu.VMEM((CH, D // 2), jnp.int32) for _ in range(2)],
        [pltpu.VMEM((CH, D), jnp.float32) for _ in range(2)],
        pltpu.VMEM_SHARED((N_pad, D), jnp.float32),
        [pltpu.SemaphoreType.DMA for _ in range(4)],
        [pltpu.SemaphoreType.DMA for _ in range(2)],
        [pltpu.SemaphoreType.DMA for _ in range(2)],
        [pltpu.SemaphoreType.DMA for _ in range(2)],
    ]

    def body(nf_hbm, we_hbm, src_hbm, dst_hbm, zrows_hbm, agg_out,
             sidx, didx, rows, web, prod, agg_sp, isem, gsem, wsem, ssem):
        c = lax.axis_index("c")
        s = lax.axis_index("s")
        wid = s * NC + c
        base = wid * nchk

        pltpu.sync_copy(zrows_hbm.at[pl.ds(s * rps, rps)],
                        agg_sp.at[pl.ds(s * rps, rps)])
        plsc.subcore_barrier()

        def start_idx(j, r):
            pltpu.async_copy(src_hbm.at[base + j], sidx[r], isem[r])
            pltpu.async_copy(dst_hbm.at[base + j], didx[r], isem[r])

        def wait_idx(j, r):
            pltpu.make_async_copy(src_hbm.at[base + j], sidx[r], isem[r]).wait()
            pltpu.make_async_copy(dst_hbm.at[base + j], didx[r], isem[r]).wait()

        def start_gw(j, r, q):
            pltpu.async_copy(nf_hbm.at[sidx[r]], rows[q], gsem[q])
            pltpu.async_copy(we_hbm.at[base + j], web[q], wsem[q])

        def wait_gw(j, r, q):
            pltpu.make_async_copy(nf_hbm.at[sidx[r]], rows[q], gsem[q]).wait()
            pltpu.make_async_copy(we_hbm.at[base + j], web[q], wsem[q]).wait()

        himask = jnp.int32(-65536)

        def mult(q):
            def mul(j, _):
                rr = j // (D // 32)
                c16 = (j % (D // 32)) * LANES
                bc = lambda v: lax.bitcast_convert_type(v, jnp.float32)
                wi = web[q][rr, pl.ds(c16, LANES)]
                wa = bc(lax.shift_left(wi, 16))
                wb = bc(lax.bitwise_and(wi, himask))
                ra = rows[q][rr, pl.ds(c16, LANES)]
                rb = rows[q][rr, pl.ds(c16 + D // 2, LANES)]
                prod[q][rr, pl.ds(c16, LANES)] = ra * wa
                prod[q][rr, pl.ds(c16 + D // 2, LANES)] = rb * wb
                return 0
            lax.fori_loop(0, CH * (D // 32), mul, 0, unroll=8)

        def start_scat(r, q):
            pltpu.async_copy(prod[q], agg_sp.at[didx[r]], ssem[q], add=True)

        def wait_scat(r, q):
            pltpu.make_async_copy(prod[q], agg_sp.at[didx[r]], ssem[q]).wait()

        start_idx(0, 0)
        start_idx(1, 1)
        wait_idx(0, 0)
        start_gw(0, 0, 0)
        start_idx(2, 2)
        wait_gw(0, 0, 0)
        wait_idx(1, 1)
        start_gw(1, 1, 1)
        mult(0)
        start_scat(0, 0)
        start_idx(3, 3)
        wait_gw(1, 1, 1)
        wait_idx(2, 2)
        start_gw(2, 2, 0)
        mult(1)
        start_scat(1, 1)

        def quad_body(i, carry):
            for rr in range(4):
                j = 4 * i + 2 + rr
                slot = (2 + rr) % 4
                q = rr % 2
                wait_scat((slot + 2) % 4, q)
                start_idx(j + 2, (slot + 2) % 4)
                wait_gw(j, slot, q)
                wait_idx(j + 1, (slot + 1) % 4)
                start_gw(j + 1, (slot + 1) % 4, 1 - q)
                mult(q)
                start_scat(slot, q)
            return carry

        lax.fori_loop(0, (nchk - 4) // 4, quad_body, 0)
        wait_scat(0, 0)
        wait_gw(nchk - 2, 2, 0)
        wait_idx(nchk - 1, 3)
        start_gw(nchk - 1, 3, 1)
        mult(0)
        start_scat(2, 0)
        wait_scat(1, 1)
        wait_gw(nchk - 1, 3, 1)
        mult(1)
        start_scat(3, 1)
        wait_scat(2, 0)
        wait_scat(3, 1)
        plsc.subcore_barrier()

        pltpu.sync_copy(agg_sp.at[pl.ds(s * rps, rps)],
                        agg_out.at[c, pl.ds(s * rps, rps)])

    return pl.kernel(body, out_type=out_type, mesh=mesh, scratch_types=scratch)


def _make_count_kernel(N_pad, E_pad):
    n_chunks = E_pad // CH // NW
    rps = N_pad // NS
    mesh = plsc.VectorSubcoreMesh(core_axis_name="c", subcore_axis_name="s")

    out_type = [jax.ShapeDtypeStruct((NC, N_pad, LANES), jnp.float32)]
    scratch = [
        pltpu.VMEM((CH,), jnp.int32),
        pltpu.VMEM((CH, LANES), jnp.float32),
        pltpu.VMEM_SHARED((N_pad, LANES), jnp.float32),
    ]

    def body(dst_hbm, zcnt_hbm, ones_hbm, cnt_out, dst_v, ones_v, cnt_sp):
        c = lax.axis_index("c")
        s = lax.axis_index("s")
        wid = s * NC + c
        pltpu.sync_copy(zcnt_hbm.at[pl.ds(s * rps, rps)],
                        cnt_sp.at[pl.ds(s * rps, rps)])
        pltpu.sync_copy(ones_hbm, ones_v)
        plsc.subcore_barrier()
        base = wid * n_chunks

        def chunk(i, carry):
            pltpu.sync_copy(dst_hbm.at[base + i], dst_v)
            pltpu.sync_copy(ones_v, cnt_sp.at[dst_v], add=True)
            return carry

        lax.fori_loop(0, n_chunks, chunk, 0)
        plsc.subcore_barrier()
        pltpu.sync_copy(cnt_sp.at[pl.ds(s * rps, rps)],
                        cnt_out.at[c, pl.ds(s * rps, rps)])

    return pl.kernel(body, out_type=out_type, mesh=mesh, scratch_types=scratch)


def kernel(node_features, node_attr, edge_src, edge_dst, edge_attr,
           edge_scalars, params):
    N, D = node_features.shape
    E = edge_scalars.shape[0]
    S = edge_scalars.shape[1]
    quantum = 4 * NW * CH
    E_pad = ((E + quantum - 1) // quantum) * quantum
    N_pad = ((N + 1 + 127) // 128) * 128
    pad = E_pad - E
    if pad:
        src_p = jnp.concatenate([edge_src, jnp.zeros((pad,), jnp.int32)])
        dst_p = jnp.concatenate([edge_dst, jnp.full((pad,), N, jnp.int32)])
        es_p = jnp.concatenate([edge_scalars, jnp.zeros((pad, S), jnp.float32)])
        ea_p = jnp.concatenate([edge_attr, jnp.zeros((pad, 1), jnp.float32)])
    else:
        src_p, dst_p, es_p, ea_p = edge_src, edge_dst, edge_scalars, edge_attr

    L = len(params)

    w1stk = _stack3(jnp.concatenate([p['fc_w1'] for p in params], axis=1))
    w2stk = jnp.stack([_stack3(p['fc_w2']) for p in params])
    we_all = _edge_weights_tc(es_p, ea_p, w1stk, w2stk, L, D)

    TOT = E_pad // CH
    src2 = src_p.reshape(TOT, CH)
    dst2 = dst_p.reshape(TOT, CH)
    we3 = [w.reshape(TOT, CH, D // 2) for w in we_all]

    zrows = jnp.zeros((N_pad, D), jnp.float32)
    zcnt = jnp.zeros((N_pad, LANES), jnp.float32)
    ones_b = jnp.ones((CH, LANES), jnp.float32)

    sc_edge = _make_sc_kernel(N_pad, E_pad, D)
    (cntp,) = _make_count_kernel(N_pad, E_pad)(dst2, zcnt, ones_b)

    x = node_features
    for li, p in enumerate(params):
        nf, scv = _node_linears_tc(x, node_attr,
                                   _stack3(p['lin1_w'][:, 0, :]),
                                   _stack3(p['sc_w'][:, 0, :]))
        (aggp,) = sc_edge(nf, we3[li], src2, dst2, zrows)
        x = _combine_tc(aggp, cntp, scv, node_attr, p['lin2_w'][:, 0, :],
                        p['lin3_w'][:, 0, :], li < L - 1)
    return x


# speedup vs baseline: 1.5891x; 1.1029x over previous
import functools

import numpy as np
import jax
import jax.numpy as jnp
from jax import lax
from jax.experimental import pallas as pl
from jax.experimental.pallas import tpu as pltpu
from jax.experimental.pallas import tpu_sc as plsc

NC = 2
NS = 16
LANES = 16
NW = NC * NS
CH = 56


def _hi_lo(x):
    xh = x.astype(jnp.bfloat16)
    xl = (x - xh.astype(jnp.float32)).astype(jnp.bfloat16)
    return xh, xl


def _stack3(w):
    wh, wl = _hi_lo(w)
    return jnp.concatenate([wh, wl, wh], axis=0)


def _bf16_round_bits(x):
    b = jax.lax.bitcast_convert_type(x, jnp.int32)
    return jax.lax.shift_right_logical(b + jnp.int32(32768), 16)


def _pack_rows_bf16(w, D):
    bits = _bf16_round_bits(w)
    return jax.lax.bitwise_or(
        bits[:, :D // 2], jax.lax.shift_left(bits[:, D // 2:], 16))


def _edge_weights_tc(es, ea, w1stk, w2stk, L, D):
    E, S = es.shape
    B = 1024
    inv = 1.0 / np.sqrt(S)

    def body(es_ref, ea_ref, w1_ref, w2_ref, *out_refs):
        esb = es_ref[...]
        eab = ea_ref[...]
        esh, esl = _hi_lo(esb)
        lhs1 = jnp.concatenate([esh, esh, esl], axis=1)
        hcat = jax.nn.silu(
            jnp.dot(lhs1, w1_ref[...], preferred_element_type=jnp.float32)
            * inv)
        for l in range(L):
            hh, hl = _hi_lo(hcat[:, S * l:S * (l + 1)])
            lhs2 = jnp.concatenate([hh, hh, hl], axis=1)
            w = jnp.dot(lhs2, w2_ref[l],
                        preferred_element_type=jnp.float32) * inv
            out_refs[l][...] = _pack_rows_bf16(w * eab, D)

    return pl.pallas_call(
        body,
        grid=(E // B,),
        in_specs=[
            pl.BlockSpec((B, S), lambda e: (e, 0)),
            pl.BlockSpec((B, 1), lambda e: (e, 0)),
            pl.BlockSpec((3 * S, L * S), lambda e: (0, 0)),
            pl.BlockSpec((L, 3 * S, D), lambda e: (0, 0, 0)),
        ],
        out_specs=[pl.BlockSpec((B, D // 2), lambda e: (e, 0))] * L,
        out_shape=[jax.ShapeDtypeStruct((E, D // 2), jnp.int32)] * L,
    )(es, ea, w1stk, w2stk)


def _node_linears_tc(x, na, lin1, scw):
    N, D = x.shape
    B = 1000
    inv = 1.0 / np.sqrt(D)

    def body(x_ref, na_ref, l1_ref, sc_ref, nf_ref, scv_ref):
        xa = x_ref[...]
        naa = na_ref[...]
        xh, xl = _hi_lo(xa)
        lhs = jnp.concatenate([xh, xh, xl], axis=1)
        nf_ref[...] = jnp.dot(lhs, l1_ref[...],
                              preferred_element_type=jnp.float32) * inv * naa
        scv_ref[...] = jnp.dot(lhs, sc_ref[...],
                               preferred_element_type=jnp.float32) * inv * naa

    return pl.pallas_call(
        body,
        grid=(N // B,),
        in_specs=[
            pl.BlockSpec((B, D), lambda i: (i, 0)),
            pl.BlockSpec((B, 1), lambda i: (i, 0)),
            pl.BlockSpec((3 * D, D), lambda i: (0, 0)),
            pl.BlockSpec((3 * D, D), lambda i: (0, 0)),
        ],
        out_specs=[pl.BlockSpec((B, D), lambda i: (i, 0))] * 2,
        out_shape=[jax.ShapeDtypeStruct((N, D), jnp.float32)] * 2,
    )(x, na, lin1, scw)


def _combine_tc(aggp, cntp, scv, na, lin2, l3, apply_silu):
    N, D = scv.shape
    B = 1000
    inv = 1.0 / np.sqrt(D)

    def body(agg_ref, cnt_ref, scv_ref, na_ref, l2_ref, l3_ref, out_ref):
        cnt = cnt_ref[0, :, 0:1] + cnt_ref[1, :, 0:1]
        a = (agg_ref[0] + agg_ref[1]) / jnp.sqrt(cnt)
        naa = na_ref[...]
        co = jnp.dot(a, l2_ref[...], preferred_element_type=jnp.float32) * inv * naa
        ang = 0.1 * jnp.dot(a, l3_ref[...],
                            preferred_element_type=jnp.float32) * inv * naa
        out = jnp.cos(ang) * scv_ref[...] + jnp.sin(ang) * co
        if apply_silu:
            out = out * jax.nn.sigmoid(out)
        out_ref[...] = out

    return pl.pallas_call(
        body,
        grid=(N // B,),
        in_specs=[
            pl.BlockSpec((2, B, D), lambda i: (0, i, 0)),
            pl.BlockSpec((2, B, LANES), lambda i: (0, i, 0)),
            pl.BlockSpec((B, D), lambda i: (i, 0)),
            pl.BlockSpec((B, 1), lambda i: (i, 0)),
            pl.BlockSpec((D, D), lambda i: (0, 0)),
            pl.BlockSpec((D, 1), lambda i: (0, 0)),
        ],
        out_specs=pl.BlockSpec((B, D), lambda i: (i, 0)),
        out_shape=jax.ShapeDtypeStruct((N, D), jnp.float32),
    )(aggp, cntp, scv, na, lin2, l3)


def _make_sc_kernel(N_pad, E_pad, D):
    TOT = E_pad // CH
    nchk = TOT // NW
    nquad = nchk // 4
    rps = N_pad // NS
    mesh = plsc.VectorSubcoreMesh(core_axis_name="c", subcore_axis_name="s")

    out_type = [jax.ShapeDtypeStruct((NC, N_pad, D), jnp.float32)]
    scratch = [
        [pltpu.VMEM((CH,), jnp.int32) for _ in range(4)],
        [pltpu.VMEM((CH,), jnp.int32) for _ in range(4)],
        [pltpu.VMEM((CH, D), jnp.float32) for _ in range(2)],
        [pltpu.VMEM((CH, D // 2), jnp.int32) for _ in range(2)],
        [pltpu.VMEM((CH, D), jnp.float32) for _ in range(2)],
        pltpu.VMEM_SHARED((N_pad, D), jnp.float32),
        [pltpu.SemaphoreType.DMA for _ in range(4)],
        [pltpu.SemaphoreType.DMA for _ in range(2)],
        [pltpu.SemaphoreType.DMA for _ in range(2)],
        [pltpu.SemaphoreType.DMA for _ in range(2)],
    ]

    def body(nf_hbm, we_hbm, src_hbm, dst_hbm, zrows_hbm, agg_out,
             sidx, didx, rows, web, prod, agg_sp, isem, gsem, wsem, ssem):
        c = lax.axis_index("c")
        s = lax.axis_index("s")
        wid = s * NC + c
        base = wid * nchk

        pltpu.sync_copy(zrows_hbm.at[pl.ds(s * rps, rps)],
                        agg_sp.at[pl.ds(s * rps, rps)])
        plsc.subcore_barrier()

        def start_idx(j, r):
            pltpu.async_copy(src_hbm.at[base + j], sidx[r], isem[r])
            pltpu.async_copy(dst_hbm.at[base + j], didx[r], isem[r])

        def wait_idx(j, r):
            pltpu.make_async_copy(src_hbm.at[base + j], sidx[r], isem[r]).wait()
            pltpu.make_async_copy(dst_hbm.at[base + j], didx[r], isem[r]).wait()

        def start_gw(j, r, q):
            pltpu.async_copy(nf_hbm.at[sidx[r]], rows[q], gsem[q])
            pltpu.async_copy(we_hbm.at[base + j], web[q], wsem[q])

        def wait_gw(j, r, q):
            pltpu.make_async_copy(nf_hbm.at[sidx[r]], rows[q], gsem[q]).wait()
            pltpu.make_async_copy(we_hbm.at[base + j], web[q], wsem[q]).wait()

        himask = jnp.int32(-65536)

        def mult(q):
            def mul(j, _):
                rr = j // (D // 32)
                c16 = (j % (D // 32)) * LANES
                bc = lambda v: lax.bitcast_convert_type(v, jnp.float32)
                wi = web[q][rr, pl.ds(c16, LANES)]
                wa = bc(lax.shift_left(wi, 16))
                wb = bc(lax.bitwise_and(wi, himask))
                ra = rows[q][rr, pl.ds(c16, LANES)]
                rb = rows[q][rr, pl.ds(c16 + D // 2, LANES)]
                prod[q][rr, pl.ds(c16, LANES)] = ra * wa
                prod[q][rr, pl.ds(c16 + D // 2, LANES)] = rb * wb
                return 0
            lax.fori_loop(0, CH * (D // 32), mul, 0, unroll=8)

        def start_scat(r, q):
            pltpu.async_copy(prod[q], agg_sp.at[didx[r]], ssem[q], add=True)

        def wait_scat(r, q):
            pltpu.make_async_copy(prod[q], agg_sp.at[didx[r]], ssem[q]).wait()

        start_idx(0, 0)
        start_idx(1, 1)
        wait_idx(0, 0)
        start_gw(0, 0, 0)
        start_idx(2, 2)
        wait_gw(0, 0, 0)
        wait_idx(1, 1)
        start_gw(1, 1, 1)
        mult(0)
        start_scat(0, 0)
        start_idx(3, 3)
        wait_gw(1, 1, 1)
        wait_idx(2, 2)
        start_gw(2, 2, 0)
        mult(1)
        start_scat(1, 1)

        def quad_body(i, carry):
            for rr in range(4):
                j = 4 * i + 2 + rr
                slot = (2 + rr) % 4
                q = rr % 2
                wait_scat((slot + 2) % 4, q)
                start_idx(j + 2, (slot + 2) % 4)
                wait_gw(j, slot, q)
                wait_idx(j + 1, (slot + 1) % 4)
                start_gw(j + 1, (slot + 1) % 4, 1 - q)
                mult(q)
                start_scat(slot, q)
            return carry

        lax.fori_loop(0, (nchk - 4) // 4, quad_body, 0)
        wait_scat(0, 0)
        wait_gw(nchk - 2, 2, 0)
        wait_idx(nchk - 1, 3)
        start_gw(nchk - 1, 3, 1)
        mult(0)
        start_scat(2, 0)
        wait_scat(1, 1)
        wait_gw(nchk - 1, 3, 1)
        mult(1)
        start_scat(3, 1)
        wait_scat(2, 0)
        wait_scat(3, 1)
        plsc.subcore_barrier()

        pltpu.sync_copy(agg_sp.at[pl.ds(s * rps, rps)],
                        agg_out.at[c, pl.ds(s * rps, rps)])

    return pl.kernel(body, out_type=out_type, mesh=mesh, scratch_types=scratch)


def _make_count_kernel(N_pad, E_pad):
    n_chunks = E_pad // CH // NW
    rps = N_pad // NS
    mesh = plsc.VectorSubcoreMesh(core_axis_name="c", subcore_axis_name="s")

    out_type = [jax.ShapeDtypeStruct((NC, N_pad, LANES), jnp.float32)]
    scratch = [
        pltpu.VMEM((CH,), jnp.int32),
        pltpu.VMEM((CH, LANES), jnp.float32),
        pltpu.VMEM_SHARED((N_pad, LANES), jnp.float32),
    ]

    def body(dst_hbm, zcnt_hbm, ones_hbm, cnt_out, dst_v, ones_v, cnt_sp):
        c = lax.axis_index("c")
        s = lax.axis_index("s")
        wid = s * NC + c
        pltpu.sync_copy(zcnt_hbm.at[pl.ds(s * rps, rps)],
                        cnt_sp.at[pl.ds(s * rps, rps)])
        pltpu.sync_copy(ones_hbm, ones_v)
        plsc.subcore_barrier()
        base = wid * n_chunks

        def chunk(i, carry):
            pltpu.sync_copy(dst_hbm.at[base + i], dst_v)
            pltpu.sync_copy(ones_v, cnt_sp.at[dst_v], add=True)
            return carry

        lax.fori_loop(0, n_chunks, chunk, 0)
        plsc.subcore_barrier()
        pltpu.sync_copy(cnt_sp.at[pl.ds(s * rps, rps)],
                        cnt_out.at[c, pl.ds(s * rps, rps)])

    return pl.kernel(body, out_type=out_type, mesh=mesh, scratch_types=scratch)


def kernel(node_features, node_attr, edge_src, edge_dst, edge_attr,
           edge_scalars, params):
    N, D = node_features.shape
    E = edge_scalars.shape[0]
    S = edge_scalars.shape[1]
    quantum = 4 * NW * CH
    E_pad = ((E + quantum - 1) // quantum) * quantum
    N_pad = ((N + 1 + 127) // 128) * 128
    pad = E_pad - E
    if pad:
        src_p = jnp.concatenate([edge_src, jnp.zeros((pad,), jnp.int32)])
        dst_p = jnp.concatenate([edge_dst, jnp.full((pad,), N, jnp.int32)])
        es_p = jnp.concatenate([edge_scalars, jnp.zeros((pad, S), jnp.float32)])
        ea_p = jnp.concatenate([edge_attr, jnp.zeros((pad, 1), jnp.float32)])
    else:
        src_p, dst_p, es_p, ea_p = edge_src, edge_dst, edge_scalars, edge_attr

    L = len(params)

    w1stk = _stack3(jnp.concatenate([p['fc_w1'] for p in params], axis=1))
    w2stk = jnp.stack([_stack3(p['fc_w2']) for p in params])
    we_all = _edge_weights_tc(es_p, ea_p, w1stk, w2stk, L, D)

    TOT = E_pad // CH
    src2 = src_p.reshape(TOT, CH)
    dst2 = dst_p.reshape(TOT, CH)
    we3 = [w.reshape(TOT, CH, D // 2) for w in we_all]

    zrows = jnp.zeros((N_pad, D), jnp.float32)
    zcnt = jnp.zeros((N_pad, LANES), jnp.float32)
    ones_b = jnp.ones((CH, LANES), jnp.float32)

    sc_edge = _make_sc_kernel(N_pad, E_pad, D)
    (cntp,) = _make_count_kernel(N_pad, E_pad)(dst2, zcnt, ones_b)

    x = node_features
    for li, p in enumerate(params):
        nf, scv = _node_linears_tc(x, node_attr,
                                   _stack3(p['lin1_w'][:, 0, :]),
                                   _stack3(p['sc_w'][:, 0, :]))
        (aggp,) = sc_edge(nf, we3[li], src2, dst2, zrows)
        x = _combine_tc(aggp, cntp, scv, node_attr, p['lin2_w'][:, 0, :],
                        p['lin3_w'][:, 0, :], li < L - 1)
    return x


# speedup vs baseline: 1.6252x; 1.0227x over previous
import functools

import numpy as np
import jax
import jax.numpy as jnp
from jax import lax
from jax.experimental import pallas as pl
from jax.experimental.pallas import tpu as pltpu
from jax.experimental.pallas import tpu_sc as plsc

NC = 2
NS = 16
LANES = 16
NW = NC * NS
CH = 56


def _hi_lo(x):
    xh = x.astype(jnp.bfloat16)
    xl = (x - xh.astype(jnp.float32)).astype(jnp.bfloat16)
    return xh, xl


def _stack3(w):
    wh, wl = _hi_lo(w)
    return jnp.concatenate([wh, wl, wh], axis=0)


def _bf16_round_bits(x):
    b = jax.lax.bitcast_convert_type(x, jnp.int32)
    return jax.lax.shift_right_logical(b + jnp.int32(32768), 16)


def _pack_rows_bf16(w, D):
    bits = _bf16_round_bits(w)
    return jax.lax.bitwise_or(
        bits[:, :D // 2], jax.lax.shift_left(bits[:, D // 2:], 16))


def _edge_weights_tc(es, ea, w1stk, w2stk, L, D):
    E, S = es.shape
    B = 1280
    inv = 1.0 / np.sqrt(S)

    def body(es_ref, ea_ref, w1_ref, w2_ref, *out_refs):
        esb = es_ref[...]
        eab = ea_ref[...]
        esh, esl = _hi_lo(esb)
        lhs1 = jnp.concatenate([esh, esh, esl], axis=1)
        hcat = jax.nn.silu(
            jnp.dot(lhs1, w1_ref[...], preferred_element_type=jnp.float32)
            * inv)
        for l in range(L):
            hh, hl = _hi_lo(hcat[:, S * l:S * (l + 1)])
            lhs2 = jnp.concatenate([hh, hh, hl], axis=1)
            w = jnp.dot(lhs2, w2_ref[l],
                        preferred_element_type=jnp.float32) * inv
            out_refs[l][...] = _pack_rows_bf16(w * eab, D)

    return pl.pallas_call(
        body,
        grid=(E // B,),
        in_specs=[
            pl.BlockSpec((B, S), lambda e: (e, 0)),
            pl.BlockSpec((B, 1), lambda e: (e, 0)),
            pl.BlockSpec((3 * S, L * S), lambda e: (0, 0)),
            pl.BlockSpec((L, 3 * S, D), lambda e: (0, 0, 0)),
        ],
        out_specs=[pl.BlockSpec((B, D // 2), lambda e: (e, 0))] * L,
        out_shape=[jax.ShapeDtypeStruct((E, D // 2), jnp.int32)] * L,
    )(es, ea, w1stk, w2stk)


def _node_linears_tc(x, na, lin1, scw):
    N, D = x.shape
    B = 1000
    inv = 1.0 / np.sqrt(D)

    def body(x_ref, na_ref, l1_ref, sc_ref, nf_ref, scv_ref):
        xa = x_ref[...]
        naa = na_ref[...]
        xh, xl = _hi_lo(xa)
        lhs = jnp.concatenate([xh, xh, xl], axis=1)
        nf_ref[...] = jnp.dot(lhs, l1_ref[...],
                              preferred_element_type=jnp.float32) * inv * naa
        scv_ref[...] = jnp.dot(lhs, sc_ref[...],
                               preferred_element_type=jnp.float32) * inv * naa

    return pl.pallas_call(
        body,
        grid=(N // B,),
        in_specs=[
            pl.BlockSpec((B, D), lambda i: (i, 0)),
            pl.BlockSpec((B, 1), lambda i: (i, 0)),
            pl.BlockSpec((3 * D, D), lambda i: (0, 0)),
            pl.BlockSpec((3 * D, D), lambda i: (0, 0)),
        ],
        out_specs=[pl.BlockSpec((B, D), lambda i: (i, 0))] * 2,
        out_shape=[jax.ShapeDtypeStruct((N, D), jnp.float32)] * 2,
    )(x, na, lin1, scw)


def _combine_tc(aggp, cntp, scv, na, lin2, l3, apply_silu):
    N, D = scv.shape
    B = 1000
    inv = 1.0 / np.sqrt(D)

    def body(agg_ref, cnt_ref, scv_ref, na_ref, l2_ref, l3_ref, out_ref):
        cnt = cnt_ref[0, :, 0:1] + cnt_ref[1, :, 0:1]
        a = (agg_ref[0] + agg_ref[1]) / jnp.sqrt(cnt)
        naa = na_ref[...]
        co = jnp.dot(a, l2_ref[...], preferred_element_type=jnp.float32) * inv * naa
        ang = 0.1 * jnp.dot(a, l3_ref[...],
                            preferred_element_type=jnp.float32) * inv * naa
        out = jnp.cos(ang) * scv_ref[...] + jnp.sin(ang) * co
        if apply_silu:
            out = out * jax.nn.sigmoid(out)
        out_ref[...] = out

    return pl.pallas_call(
        body,
        grid=(N // B,),
        in_specs=[
            pl.BlockSpec((2, B, D), lambda i: (0, i, 0)),
            pl.BlockSpec((2, B, LANES), lambda i: (0, i, 0)),
            pl.BlockSpec((B, D), lambda i: (i, 0)),
            pl.BlockSpec((B, 1), lambda i: (i, 0)),
            pl.BlockSpec((D, D), lambda i: (0, 0)),
            pl.BlockSpec((D, 1), lambda i: (0, 0)),
        ],
        out_specs=pl.BlockSpec((B, D), lambda i: (i, 0)),
        out_shape=jax.ShapeDtypeStruct((N, D), jnp.float32),
    )(aggp, cntp, scv, na, lin2, l3)


def _make_sc_kernel(N_pad, E_pad, D):
    TOT = E_pad // CH
    nchk = TOT // NW
    nquad = nchk // 4
    rps = N_pad // NS
    mesh = plsc.VectorSubcoreMesh(core_axis_name="c", subcore_axis_name="s")

    out_type = [jax.ShapeDtypeStruct((NC, N_pad, D), jnp.float32)]
    scratch = [
        [pltpu.VMEM((CH,), jnp.int32) for _ in range(4)],
        [pltpu.VMEM((CH,), jnp.int32) for _ in range(4)],
        [pltpu.VMEM((CH, D), jnp.float32) for _ in range(2)],
        [pltpu.VMEM((CH, D // 2), jnp.int32) for _ in range(2)],
        [pltpu.VMEM((CH, D), jnp.float32) for _ in range(2)],
        pltpu.VMEM_SHARED((N_pad, D), jnp.float32),
        [pltpu.SemaphoreType.DMA for _ in range(4)],
        [pltpu.SemaphoreType.DMA for _ in range(2)],
        [pltpu.SemaphoreType.DMA for _ in range(2)],
        [pltpu.SemaphoreType.DMA for _ in range(2)],
    ]

    def body(nf_hbm, we_hbm, src_hbm, dst_hbm, zrows_hbm, agg_out,
             sidx, didx, rows, web, prod, agg_sp, isem, gsem, wsem, ssem):
        c = lax.axis_index("c")
        s = lax.axis_index("s")
        wid = s * NC + c
        base = wid * nchk

        pltpu.sync_copy(zrows_hbm.at[pl.ds(s * rps, rps)],
                        agg_sp.at[pl.ds(s * rps, rps)])
        plsc.subcore_barrier()

        def start_idx(j, r):
            pltpu.async_copy(src_hbm.at[base + j], sidx[r], isem[r])
            pltpu.async_copy(dst_hbm.at[base + j], didx[r], isem[r])

        def wait_idx(j, r):
            pltpu.make_async_copy(src_hbm.at[base + j], sidx[r], isem[r]).wait()
            pltpu.make_async_copy(dst_hbm.at[base + j], didx[r], isem[r]).wait()

        def start_gw(j, r, q):
            pltpu.async_copy(nf_hbm.at[sidx[r]], rows[q], gsem[q])
            pltpu.async_copy(we_hbm.at[base + j], web[q], wsem[q])

        def wait_gw(j, r, q):
            pltpu.make_async_copy(nf_hbm.at[sidx[r]], rows[q], gsem[q]).wait()
            pltpu.make_async_copy(we_hbm.at[base + j], web[q], wsem[q]).wait()

        himask = jnp.int32(-65536)

        def mult(q):
            def mul(j, _):
                rr = j // (D // 32)
                c16 = (j % (D // 32)) * LANES
                bc = lambda v: lax.bitcast_convert_type(v, jnp.float32)
                wi = web[q][rr, pl.ds(c16, LANES)]
                wa = bc(lax.shift_left(wi, 16))
                wb = bc(lax.bitwise_and(wi, himask))
                ra = rows[q][rr, pl.ds(c16, LANES)]
                rb = rows[q][rr, pl.ds(c16 + D // 2, LANES)]
                prod[q][rr, pl.ds(c16, LANES)] = ra * wa
                prod[q][rr, pl.ds(c16 + D // 2, LANES)] = rb * wb
                return 0
            lax.fori_loop(0, CH * (D // 32), mul, 0, unroll=8)

        def start_scat(r, q):
            pltpu.async_copy(prod[q], agg_sp.at[didx[r]], ssem[q], add=True)

        def wait_scat(r, q):
            pltpu.make_async_copy(prod[q], agg_sp.at[didx[r]], ssem[q]).wait()

        start_idx(0, 0)
        start_idx(1, 1)
        wait_idx(0, 0)
        start_gw(0, 0, 0)
        start_idx(2, 2)
        wait_gw(0, 0, 0)
        wait_idx(1, 1)
        start_gw(1, 1, 1)
        mult(0)
        start_scat(0, 0)
        start_idx(3, 3)
        wait_gw(1, 1, 1)
        wait_idx(2, 2)
        start_gw(2, 2, 0)
        mult(1)
        start_scat(1, 1)

        def quad_body(i, carry):
            for rr in range(4):
                j = 4 * i + 2 + rr
                slot = (2 + rr) % 4
                q = rr % 2
                wait_scat((slot + 2) % 4, q)
                start_idx(j + 2, (slot + 2) % 4)
                wait_gw(j, slot, q)
                wait_idx(j + 1, (slot + 1) % 4)
                start_gw(j + 1, (slot + 1) % 4, 1 - q)
                mult(q)
                start_scat(slot, q)
            return carry

        lax.fori_loop(0, (nchk - 4) // 4, quad_body, 0)
        wait_scat(0, 0)
        wait_gw(nchk - 2, 2, 0)
        wait_idx(nchk - 1, 3)
        start_gw(nchk - 1, 3, 1)
        mult(0)
        start_scat(2, 0)
        wait_scat(1, 1)
        wait_gw(nchk - 1, 3, 1)
        mult(1)
        start_scat(3, 1)
        wait_scat(2, 0)
        wait_scat(3, 1)
        plsc.subcore_barrier()

        pltpu.sync_copy(agg_sp.at[pl.ds(s * rps, rps)],
                        agg_out.at[c, pl.ds(s * rps, rps)])

    return pl.kernel(body, out_type=out_type, mesh=mesh, scratch_types=scratch)


def _make_count_kernel(N_pad, E_pad):
    n_chunks = E_pad // CH // NW
    rps = N_pad // NS
    mesh = plsc.VectorSubcoreMesh(core_axis_name="c", subcore_axis_name="s")

    out_type = [jax.ShapeDtypeStruct((NC, N_pad, LANES), jnp.float32)]
    scratch = [
        pltpu.VMEM((CH,), jnp.int32),
        pltpu.VMEM((CH, LANES), jnp.float32),
        pltpu.VMEM_SHARED((N_pad, LANES), jnp.float32),
    ]

    def body(dst_hbm, zcnt_hbm, ones_hbm, cnt_out, dst_v, ones_v, cnt_sp):
        c = lax.axis_index("c")
        s = lax.axis_index("s")
        wid = s * NC + c
        pltpu.sync_copy(zcnt_hbm.at[pl.ds(s * rps, rps)],
                        cnt_sp.at[pl.ds(s * rps, rps)])
        pltpu.sync_copy(ones_hbm, ones_v)
        plsc.subcore_barrier()
        base = wid * n_chunks

        def chunk(i, carry):
            pltpu.sync_copy(dst_hbm.at[base + i], dst_v)
            pltpu.sync_copy(ones_v, cnt_sp.at[dst_v], add=True)
            return carry

        lax.fori_loop(0, n_chunks, chunk, 0)
        plsc.subcore_barrier()
        pltpu.sync_copy(cnt_sp.at[pl.ds(s * rps, rps)],
                        cnt_out.at[c, pl.ds(s * rps, rps)])

    return pl.kernel(body, out_type=out_type, mesh=mesh, scratch_types=scratch)


def kernel(node_features, node_attr, edge_src, edge_dst, edge_attr,
           edge_scalars, params):
    N, D = node_features.shape
    E = edge_scalars.shape[0]
    S = edge_scalars.shape[1]
    quantum = 4 * NW * CH
    E_pad = ((E + quantum - 1) // quantum) * quantum
    N_pad = ((N + 1 + 127) // 128) * 128
    pad = E_pad - E
    if pad:
        src_p = jnp.concatenate([edge_src, jnp.zeros((pad,), jnp.int32)])
        dst_p = jnp.concatenate([edge_dst, jnp.full((pad,), N, jnp.int32)])
        es_p = jnp.concatenate([edge_scalars, jnp.zeros((pad, S), jnp.float32)])
        ea_p = jnp.concatenate([edge_attr, jnp.zeros((pad, 1), jnp.float32)])
    else:
        src_p, dst_p, es_p, ea_p = edge_src, edge_dst, edge_scalars, edge_attr

    L = len(params)

    w1stk = _stack3(jnp.concatenate([p['fc_w1'] for p in params], axis=1))
    w2stk = jnp.stack([_stack3(p['fc_w2']) for p in params])
    we_all = _edge_weights_tc(es_p, ea_p, w1stk, w2stk, L, D)

    TOT = E_pad // CH
    src2 = src_p.reshape(TOT, CH)
    dst2 = dst_p.reshape(TOT, CH)
    we3 = [w.reshape(TOT, CH, D // 2) for w in we_all]

    zrows = jnp.zeros((N_pad, D), jnp.float32)
    zcnt = jnp.zeros((N_pad, LANES), jnp.float32)
    ones_b = jnp.ones((CH, LANES), jnp.float32)

    sc_edge = _make_sc_kernel(N_pad, E_pad, D)
    (cntp,) = _make_count_kernel(N_pad, E_pad)(dst2, zcnt, ones_b)

    x = node_features
    for li, p in enumerate(params):
        nf, scv = _node_linears_tc(x, node_attr,
                                   _stack3(p['lin1_w'][:, 0, :]),
                                   _stack3(p['sc_w'][:, 0, :]))
        (aggp,) = sc_edge(nf, we3[li], src2, dst2, zrows)
        x = _combine_tc(aggp, cntp, scv, node_attr, p['lin2_w'][:, 0, :],
                        p['lin3_w'][:, 0, :], li < L - 1)
    return x
